# baseline (speedup 1.0000x reference)
import functools

import jax
import jax.numpy as jnp
from jax import lax
from jax.experimental import pallas as pl
from jax.experimental.pallas import tpu as pltpu
from jax.experimental.pallas import tpu_sc as plsc


def _scores_kernel(q_ref, w0_ref, b0_ref, w1_ref, b1_ref, o_ref, w0r_s, b0c_s, ewt_s, *, num_types):
    neurons = w0_ref.shape[2]
    th = num_types * neurons

    @pl.when(pl.program_id(0) == 0)
    def _prep():
        w0r_s[...] = jnp.concatenate([w0_ref[t] for t in range(num_types)], axis=1)
        b0row = jnp.concatenate([b0_ref[t:t + 1, :] for t in range(num_types)], axis=1)
        b0c_s[...] = jnp.transpose(b0row)
        c_iota = jax.lax.broadcasted_iota(jnp.int32, (num_types, th), 1)
        r_iota = jax.lax.broadcasted_iota(jnp.int32, (num_types, th), 0)
        w1tile = jnp.tile(w1_ref[...], (1, num_types))
        ewt_s[...] = jnp.where(c_iota // neurons == r_iota, w1tile, 0.0)

    qb = q_ref[...]
    pt = jax.lax.dot_general(w0r_s[...], qb, (((0,), (1,)), ((), ())),
                             preferred_element_type=jnp.float32)
    ht = jnp.tanh(pt + b0c_s[...])
    st = jnp.dot(ewt_s[...], ht, preferred_element_type=jnp.float32)
    o_ref[...] = st + b1_ref[0, 0]


def _make_select(n, num_types):
    info = plsc.get_sparse_core_info()
    nc, ns, lanes = info.num_cores, info.num_subcores, info.num_lanes
    nw = nc * ns
    per_w = n // nw
    mesh = plsc.VectorSubcoreMesh(core_axis_name="c", subcore_axis_name="s")

    @functools.partial(
        pl.kernel, mesh=mesh,
        out_type=jax.ShapeDtypeStruct((n,), jnp.float32),
        scratch_types=[
            pltpu.VMEM((num_types * per_w,), jnp.float32),
            pltpu.VMEM((per_w,), jnp.int32),
            pltpu.VMEM((per_w,), jnp.float32),
        ],
    )
    def select(st_hbm, z_hbm, out_hbm, slab_v, z_v, f_v):
        wid = lax.axis_index("s") * nc + lax.axis_index("c")
        base = wid * per_w
        for t in range(num_types):
            pltpu.sync_copy(st_hbm.at[t, pl.ds(base, per_w)],
                            slab_v.at[pl.ds(t * per_w, per_w)])
        pltpu.sync_copy(z_hbm.at[pl.ds(base, per_w)], z_v)
        for i in range(per_w // lanes):
            zi = z_v[pl.ds(i * lanes, lanes)]
            acc = slab_v[pl.ds(i * lanes, lanes)]
            for t in range(1, num_types):
                row = slab_v[pl.ds(t * per_w + i * lanes, lanes)]
                acc = jnp.where(zi == t, row, acc)
            f_v[pl.ds(i * lanes, lanes)] = acc
        pltpu.sync_copy(f_v, out_hbm.at[pl.ds(base, per_w)])

    return select


def kernel(q, Z, W0, b0, W1, b1):
    n, d = q.shape
    num_types, _, neurons = W0.shape
    th = num_types * neurons
    blk = 4096
    grid = n // blk

    b1a = jnp.full((1, 1), b1, dtype=jnp.float32)

    st = pl.pallas_call(
        functools.partial(_scores_kernel, num_types=num_types),
        grid=(grid,),
        in_specs=[
            pl.BlockSpec((blk, d), lambda i: (i, 0)),
            pl.BlockSpec((num_types, d, neurons), lambda i: (0, 0, 0)),
            pl.BlockSpec((num_types, neurons), lambda i: (0, 0)),
            pl.BlockSpec((num_types, neurons), lambda i: (0, 0)),
            pl.BlockSpec((1, 1), lambda i: (0, 0)),
        ],
        out_specs=pl.BlockSpec((num_types, blk), lambda i: (0, i)),
        out_shape=jax.ShapeDtypeStruct((num_types, n), jnp.float32),
        scratch_shapes=[
            pltpu.VMEM((d, th), jnp.float32),
            pltpu.VMEM((th, 1), jnp.float32),
            pltpu.VMEM((num_types, th), jnp.float32),
        ],
    )(q, W0, b0, W1, b1a)

    return _make_select(n, num_types)(st, Z)


# speedup vs baseline: 2.8525x; 2.8525x over previous
import functools

import jax
import jax.numpy as jnp
from jax.experimental import pallas as pl
from jax.experimental.pallas import tpu as pltpu


def _pipeline_kernel(q_hbm, z_ref, w0_ref, b0_ref, w1_ref, b1_ref, o_ref,
                     qbuf, sem, *, num_types, chunk, nchunks):
    neurons = w0_ref.shape[2]
    th = num_types * neurons

    def copy_in(c, slot):
        return pltpu.make_async_copy(
            q_hbm.at[pl.ds(c * chunk, chunk), :], qbuf.at[slot], sem.at[slot])

    copy_in(0, 0).start()
    copy_in(1, 1).start()

    w0r = jnp.concatenate([w0_ref[t] for t in range(num_types)], axis=1)
    w0rb = w0r.astype(jnp.bfloat16)
    b0row = jnp.concatenate([b0_ref[t:t + 1, :] for t in range(num_types)], axis=1)
    b0c = jnp.transpose(b0row)
    c_iota = jax.lax.broadcasted_iota(jnp.int32, (num_types, th), 1)
    r_iota = jax.lax.broadcasted_iota(jnp.int32, (num_types, th), 0)
    w1tile = jnp.tile(w1_ref[...], (1, num_types))
    ewt = jnp.where(c_iota // neurons == r_iota, w1tile, 0.0)
    b1 = b1_ref[0, 0]

    for c in range(nchunks):
        slot = c % 2
        copy_in(c, slot).wait()
        qb = qbuf[slot]
        pt = jax.lax.dot_general(w0rb, qb.astype(jnp.bfloat16),
                                 (((0,), (1,)), ((), ())),
                                 preferred_element_type=jnp.float32)
        ht = jnp.tanh(pt + b0c)
        st = jnp.dot(ewt, ht, preferred_element_type=jnp.float32)
        t_iota = jax.lax.broadcasted_iota(jnp.int32, (num_types, chunk), 0)
        sel = jnp.where(t_iota == z_ref[pl.ds(c * chunk, chunk)][None, :], st, 0.0)
        o_ref[pl.ds(c * chunk, chunk)] = jnp.sum(sel, axis=0) + b1
        if c + 2 < nchunks:
            copy_in(c + 2, slot).start()


def kernel(q, Z, W0, b0, W1, b1):
    n, d = q.shape
    num_types, _, neurons = W0.shape
    chunk = 1024
    nchunks = n // chunk

    b1a = jnp.full((1, 1), b1, dtype=jnp.float32)

    f = pl.pallas_call(
        functools.partial(_pipeline_kernel, num_types=num_types,
                          chunk=chunk, nchunks=nchunks),
        in_specs=[
            pl.BlockSpec(memory_space=pltpu.MemorySpace.HBM),
            pl.BlockSpec(memory_space=pltpu.MemorySpace.VMEM),
            pl.BlockSpec(memory_space=pltpu.MemorySpace.VMEM),
            pl.BlockSpec(memory_space=pltpu.MemorySpace.VMEM),
            pl.BlockSpec(memory_space=pltpu.MemorySpace.VMEM),
            pl.BlockSpec(memory_space=pltpu.MemorySpace.VMEM),
        ],
        out_specs=pl.BlockSpec(memory_space=pltpu.MemorySpace.VMEM),
        out_shape=jax.ShapeDtypeStruct((n,), jnp.float32),
        scratch_shapes=[
            pltpu.VMEM((2, chunk, d), jnp.float32),
            pltpu.SemaphoreType.DMA((2,)),
        ],
    )(q, Z, W0, b0, W1, b1a)

    return f


# speedup vs baseline: 3.2869x; 1.1523x over previous
import functools

import jax
import jax.numpy as jnp
from jax.experimental import pallas as pl
from jax.experimental.pallas import tpu as pltpu


def _mlp_block_kernel(q_ref, z_ref, w0_ref, b0_ref, w1_ref, b1_ref, o_ref,
                      w0r_s, b0c_s, ewt_s, *, num_types):
    neurons = w0_ref.shape[2]
    th = num_types * neurons

    @pl.when(pl.program_id(0) == 0)
    def _prep():
        w0r_s[...] = jnp.concatenate([w0_ref[t] for t in range(num_types)], axis=1)
        b0row = jnp.concatenate([b0_ref[t:t + 1, :] for t in range(num_types)], axis=1)
        b0c_s[...] = jnp.transpose(b0row)
        c_iota = jax.lax.broadcasted_iota(jnp.int32, (num_types, th), 1)
        r_iota = jax.lax.broadcasted_iota(jnp.int32, (num_types, th), 0)
        w1tile = jnp.tile(w1_ref[...], (1, num_types))
        ewt_s[...] = jnp.where(c_iota // neurons == r_iota, w1tile, 0.0)

    qb = q_ref[...]
    blk = qb.shape[0]
    pt = jax.lax.dot_general(w0r_s[...].astype(jnp.bfloat16), qb.astype(jnp.bfloat16),
                             (((0,), (1,)), ((), ())),
                             preferred_element_type=jnp.float32)
    ht = jnp.tanh(pt + b0c_s[...])
    st = jnp.dot(ewt_s[...], ht, preferred_element_type=jnp.float32)
    t_iota = jax.lax.broadcasted_iota(jnp.int32, (num_types, blk), 0)
    sel = jnp.where(t_iota == z_ref[...][None, :], st, 0.0)
    o_ref[...] = jnp.sum(sel, axis=0) + b1_ref[0, 0]


def kernel(q, Z, W0, b0, W1, b1):
    n, d = q.shape
    num_types, _, neurons = W0.shape
    th = num_types * neurons
    blk = 4096
    grid = n // blk

    b1a = jnp.full((1, 1), b1, dtype=jnp.float32)

    f = pl.pallas_call(
        functools.partial(_mlp_block_kernel, num_types=num_types),
        grid=(grid,),
        in_specs=[
            pl.BlockSpec((blk, d), lambda i: (i, 0)),
            pl.BlockSpec((blk,), lambda i: (i,)),
            pl.BlockSpec((num_types, d, neurons), lambda i: (0, 0, 0)),
            pl.BlockSpec((num_types, neurons), lambda i: (0, 0)),
            pl.BlockSpec((num_types, neurons), lambda i: (0, 0)),
            pl.BlockSpec((1, 1), lambda i: (0, 0)),
        ],
        out_specs=pl.BlockSpec((blk,), lambda i: (i,)),
        out_shape=jax.ShapeDtypeStruct((n,), jnp.float32),
        scratch_shapes=[
            pltpu.VMEM((d, th), jnp.float32),
            pltpu.VMEM((th, 1), jnp.float32),
            pltpu.VMEM((num_types, th), jnp.float32),
        ],
    )(q, Z, W0, b0, W1, b1a)

    return f


# speedup vs baseline: 3.3006x; 1.0042x over previous
import functools

import jax
import jax.numpy as jnp
from jax.experimental import pallas as pl
from jax.experimental.pallas import tpu as pltpu


def _mlp_block_kernel(q_ref, z_ref, w0_ref, b0_ref, w1_ref, b1_ref, o_ref,
                      w0r_s, b0c_s, ewt_s, *, num_types):
    neurons = w0_ref.shape[2]
    th = num_types * neurons

    @pl.when(pl.program_id(0) == 0)
    def _prep():
        w0r_s[...] = jnp.concatenate([w0_ref[t] for t in range(num_types)], axis=1)
        b0row = jnp.concatenate([b0_ref[t:t + 1, :] for t in range(num_types)], axis=1)
        b0c_s[...] = jnp.transpose(b0row)
        c_iota = jax.lax.broadcasted_iota(jnp.int32, (num_types, th), 1)
        r_iota = jax.lax.broadcasted_iota(jnp.int32, (num_types, th), 0)
        w1tile = jnp.tile(w1_ref[...], (1, num_types))
        ewt_s[...] = jnp.where(c_iota // neurons == r_iota, w1tile, 0.0)

    qb = q_ref[...]
    blk = qb.shape[0]
    pt = jax.lax.dot_general(w0r_s[...].astype(jnp.bfloat16), qb.astype(jnp.bfloat16),
                             (((0,), (1,)), ((), ())),
                             preferred_element_type=jnp.float32)
    ht = jnp.tanh(pt + b0c_s[...])
    st = jnp.dot(ewt_s[...], ht, preferred_element_type=jnp.float32)
    t_iota = jax.lax.broadcasted_iota(jnp.int32, (num_types, blk), 0)
    sel = jnp.where(t_iota == z_ref[...][None, :], st, 0.0)
    o_ref[...] = jnp.sum(sel, axis=0) + b1_ref[0, 0]


def kernel(q, Z, W0, b0, W1, b1):
    n, d = q.shape
    num_types, _, neurons = W0.shape
    th = num_types * neurons
    blk = 4096
    grid = n // blk

    b1a = jnp.full((1, 1), b1, dtype=jnp.float32)

    f = pl.pallas_call(
        functools.partial(_mlp_block_kernel, num_types=num_types),
        grid=(grid,),
        in_specs=[
            pl.BlockSpec((blk, d), lambda i: (i, 0)),
            pl.BlockSpec((blk,), lambda i: (i,)),
            pl.BlockSpec((num_types, d, neurons), lambda i: (0, 0, 0)),
            pl.BlockSpec((num_types, neurons), lambda i: (0, 0)),
            pl.BlockSpec((num_types, neurons), lambda i: (0, 0)),
            pl.BlockSpec((1, 1), lambda i: (0, 0)),
        ],
        out_specs=pl.BlockSpec((blk,), lambda i: (i,)),
        out_shape=jax.ShapeDtypeStruct((n,), jnp.float32),
        compiler_params=pltpu.CompilerParams(
            allow_input_fusion=[False, False, False, False, False, True]),
        scratch_shapes=[
            pltpu.VMEM((d, th), jnp.float32),
            pltpu.VMEM((th, 1), jnp.float32),
            pltpu.VMEM((num_types, th), jnp.float32),
        ],
    )(q, Z, W0, b0, W1, b1a)

    return f
